# baseline (speedup 1.0000x reference)
import functools

import jax
import jax.numpy as jnp
from jax import lax
from jax.experimental import pallas as pl
from jax.experimental.pallas import tpu as pltpu
from jax.experimental.pallas import tpu_sc as plsc

N = 100000
E = 1600000
K = 128
PAD_ROWS = 12800
EPAD = PAD_ROWS * K
DEPTH = 4
NF = 100096
BN = 4000
GRID = N // BN


def _colmax_body(nsteps, x_ref, o_ref):
  i = pl.program_id(0)

  @pl.when(i == 0)
  def _():
    o_ref[...] = jnp.zeros_like(o_ref)

  m = jnp.max(jnp.abs(x_ref[...]), axis=0, keepdims=True)
  m128 = jnp.concatenate(
      [jnp.broadcast_to(m, (8, 5)), jnp.zeros((8, 123), jnp.float32)], axis=1)
  acc = jnp.maximum(o_ref[...], m128)
  o_ref[...] = acc

  @pl.when(i == nsteps - 1)
  def _():
    full = jnp.max(acc, axis=0, keepdims=True)
    o_ref[...] = jnp.broadcast_to(full, (8, 128))


def _colmax(rf):
  return pl.pallas_call(
      functools.partial(_colmax_body, GRID),
      grid=(GRID,),
      in_specs=[pl.BlockSpec((BN, 5), lambda i: (i, 0))],
      out_specs=pl.BlockSpec((8, 128), lambda i: (0, 0)),
      out_shape=jax.ShapeDtypeStruct((8, 128), jnp.float32),
  )(rf)


def _layer1_pre_body(rf_ref, cat_ref, cmax_ref, emb_ref,
                     hw_ref, hb_ref, m1w_ref, m1b_ref, m2w_ref, m2b_ref,
                     hid_ref, msg4_ref):
  scale = 1.0 / (cmax_ref[0:1, 0:5] + 1e-12)
  rfn = rf_ref[...] * scale
  cat = cat_ref[...]
  iota = lax.broadcasted_iota(jnp.int32, (BN, 16), 1)
  onehot = (iota == cat).astype(jnp.float32)
  emb = jnp.dot(onehot, emb_ref[...],
                preferred_element_type=jnp.float32)
  x = jnp.concatenate([rfn, emb], axis=1)
  hid = jax.nn.relu(jnp.dot(x, hw_ref[...],
                            preferred_element_type=jnp.float32) + hb_ref[...])
  t = jax.nn.relu(jnp.dot(hid, m1w_ref[...],
                          preferred_element_type=jnp.float32) + m1b_ref[...])
  msg = jax.nn.relu(jnp.dot(t, m2w_ref[...],
                            preferred_element_type=jnp.float32) + m2b_ref[...])
  hid_ref[...] = hid
  for g in range(4):
    msg4_ref[g, :, :] = msg[:, 16 * g:16 * (g + 1)]


def _layer1_pre(rf, cat, cmax, emb0, hw, hb, m1w, m1b, m2w, m2b):
  full = lambda a: pl.BlockSpec(a.shape, lambda i: tuple(0 for _ in a.shape))
  return pl.pallas_call(
      _layer1_pre_body,
      grid=(GRID,),
      in_specs=[
          pl.BlockSpec((BN, 5), lambda i: (i, 0)),
          pl.BlockSpec((BN, 1), lambda i: (i, 0)),
          full(cmax), full(emb0),
          full(hw), full(hb), full(m1w), full(m1b), full(m2w), full(m2b),
      ],
      out_specs=[
          pl.BlockSpec((BN, 64), lambda i: (i, 0)),
          pl.BlockSpec((4, BN, 16), lambda i: (0, i, 0)),
      ],
      out_shape=[
          jax.ShapeDtypeStruct((N, 64), jnp.float32),
          jax.ShapeDtypeStruct((4, N, 16), jnp.float32),
      ],
  )(rf, cat, cmax, emb0, hw, hb, m1w, m1b, m2w, m2b)


def _layer1_post_body(f_ref, hid_ref,
                      a1w_ref, a1b_ref, a2w_ref, a2b_ref,
                      hw_ref, hb_ref, m1w_ref, m1b_ref, m2w_ref, m2b_ref,
                      hid2_ref, msg2_ref):
  f = jnp.concatenate([f_ref[g] for g in range(4)], axis=1)
  t = jax.nn.relu(jnp.dot(f, a1w_ref[...],
                          preferred_element_type=jnp.float32) + a1b_ref[...])
  agg = jax.nn.relu(jnp.dot(t, a2w_ref[...],
                            preferred_element_type=jnp.float32) + a2b_ref[...])
  x2 = agg + hid_ref[...]
  hid2 = jax.nn.relu(jnp.dot(x2, hw_ref[...],
                             preferred_element_type=jnp.float32) + hb_ref[...])
  t2 = jax.nn.relu(jnp.dot(hid2, m1w_ref[...],
                           preferred_element_type=jnp.float32) + m1b_ref[...])
  msg2 = jax.nn.relu(jnp.dot(t2, m2w_ref[...],
                             preferred_element_type=jnp.float32) + m2b_ref[...])
  hid2_ref[...] = hid2
  msg2_ref[...] = msg2


def _layer1_post(f1, hid1, a1w, a1b, a2w, a2b, hw, hb, m1w, m1b, m2w, m2b):
  full = lambda a: pl.BlockSpec(a.shape, lambda i: tuple(0 for _ in a.shape))
  return pl.pallas_call(
      _layer1_post_body,
      grid=(GRID,),
      in_specs=[
          pl.BlockSpec((4, BN, 16), lambda i: (0, i, 0)),
          pl.BlockSpec((BN, 64), lambda i: (i, 0)),
          full(a1w), full(a1b), full(a2w), full(a2b),
          full(hw), full(hb), full(m1w), full(m1b), full(m2w), full(m2b),
      ],
      out_specs=[
          pl.BlockSpec((BN, 16), lambda i: (i, 0)),
          pl.BlockSpec((BN, 16), lambda i: (i, 0)),
      ],
      out_shape=[
          jax.ShapeDtypeStruct((N, 16), jnp.float32),
          jax.ShapeDtypeStruct((N, 16), jnp.float32),
      ],
  )(f1, hid1, a1w, a1b, a2w, a2b, hw, hb, m1w, m1b, m2w, m2b)


def _layer2_post_body(f_ref, hid_ref, a1w_ref, a1b_ref, a2w_ref, a2b_ref,
                      out_ref):
  f = f_ref[0] + f_ref[1]
  t = jax.nn.relu(jnp.dot(f, a1w_ref[...],
                          preferred_element_type=jnp.float32) + a1b_ref[...])
  agg = jax.nn.relu(jnp.dot(t, a2w_ref[...],
                            preferred_element_type=jnp.float32) + a2b_ref[...])
  out_ref[...] = agg + hid_ref[...]


def _layer2_post(f2p, hid2, a1w, a1b, a2w, a2b):
  full = lambda a: pl.BlockSpec(a.shape, lambda i: tuple(0 for _ in a.shape))
  return pl.pallas_call(
      _layer2_post_body,
      grid=(GRID,),
      in_specs=[
          pl.BlockSpec((2, BN, 16), lambda i: (0, i, 0)),
          pl.BlockSpec((BN, 16), lambda i: (i, 0)),
          full(a1w), full(a1b), full(a2w), full(a2b),
      ],
      out_specs=pl.BlockSpec((BN, 16), lambda i: (i, 0)),
      out_shape=jax.ShapeDtypeStruct((N, 16), jnp.float32),
  )(f2p, hid2, a1w, a1b, a2w, a2b)


def _make_segsum(n_out, n_msg, tasks_per_sc, rows_per_tile, idxb):
  n_sb = rows_per_tile // idxb
  mesh = plsc.VectorSubcoreMesh(core_axis_name="c", subcore_axis_name="s")

  @functools.partial(
      pl.kernel,
      out_type=jax.ShapeDtypeStruct((n_out, NF, 16), jnp.float32),
      mesh=mesh,
      compiler_params=pltpu.CompilerParams(use_tc_tiling_on_sc=False),
      scratch_types=[
          pltpu.VMEM_SHARED((NF, 16), jnp.float32),
          pltpu.VMEM((idxb, K), jnp.int32),
          pltpu.VMEM((idxb, K), jnp.int32),
          pltpu.VMEM((idxb, K), jnp.int32),
          pltpu.VMEM((DEPTH, K, 16), jnp.float32),
          pltpu.SemaphoreType.DMA,
      ],
  )
  def seg(src_hbm, dst_hbm, msg_hbm, zeros_hbm, out_hbm,
          fbuf, sidx, didx, gidx, rows, sem):
    c = lax.axis_index("c")
    s = lax.axis_index("s")
    zchunk = NF // 16
    for t in range(tasks_per_sc):
      if n_msg == 1:
        slot = c
        row0 = (c * 16 + s) * rows_per_tile
      else:
        slot = 2 * c + t
        row0 = s * rows_per_tile

      pltpu.sync_copy(zeros_hbm.at[pl.ds(s * zchunk, zchunk)],
                      fbuf.at[pl.ds(s * zchunk, zchunk)])
      plsc.subcore_barrier()

      @pl.loop(0, n_sb)
      def _(sb):
        rb = row0 + sb * idxb
        pltpu.sync_copy(src_hbm.at[pl.ds(rb, idxb)], sidx)
        pltpu.sync_copy(dst_hbm.at[pl.ds(rb, idxb)], didx)

        def issue(j):
          if n_msg > 1:
            for q in range(K // 16):
              v = sidx[j, pl.ds(q * 16, 16)]
              gidx[j, pl.ds(q * 16, 16)] = v + slot * N
            idxrow = gidx.at[j]
          else:
            idxrow = sidx.at[j]
          return pltpu.async_copy(msg_hbm.at[idxrow], rows.at[j % DEPTH], sem)

        cps = [issue(j) for j in range(min(DEPTH - 1, idxb))]
        for j in range(idxb):
          if j + DEPTH - 1 < idxb:
            cps.append(issue(j + DEPTH - 1))
          cps[j].wait()
          pltpu.sync_copy(rows.at[j % DEPTH], fbuf.at[didx.at[j]], add=True)

      plsc.subcore_barrier()
      pltpu.sync_copy(fbuf.at[pl.ds(s * zchunk, zchunk)],
                      out_hbm.at[slot].at[pl.ds(s * zchunk, zchunk)])
      plsc.subcore_barrier()

  return seg


_make_segsum = functools.cache(_make_segsum)


def _segsum_l1(*args):
  return _make_segsum(n_out=4, n_msg=4, tasks_per_sc=2,
                      rows_per_tile=PAD_ROWS // 16, idxb=32)(*args)


def _segsum_l2(*args):
  return _make_segsum(n_out=2, n_msg=1, tasks_per_sc=1,
                      rows_per_tile=PAD_ROWS // 32, idxb=16)(*args)


def kernel(real_features, cat_features, edge_index, emb0,
           l1_hW, l1_hb, l1_m1W, l1_m1b, l1_m2W, l1_m2b,
           l1_a1W, l1_a1b, l1_a2W, l1_a2b,
           l2_hW, l2_hb, l2_m1W, l2_m1b, l2_m2W, l2_m2b,
           l2_a1W, l2_a1b, l2_a2W, l2_a2b):
  row2 = lambda b: b.reshape(1, -1)

  src = edge_index[0].astype(jnp.int32)
  dst = edge_index[1].astype(jnp.int32)
  pad = EPAD - E
  src2d = jnp.concatenate([src, jnp.zeros((pad,), jnp.int32)]).reshape(
      PAD_ROWS, K)
  dst2d = jnp.concatenate([dst, jnp.full((pad,), N, jnp.int32)]).reshape(
      PAD_ROWS, K)
  zeros_nf = jnp.zeros((NF, 16), jnp.float32)

  cmax = _colmax(real_features)
  hid1, msg1 = _layer1_pre(
      real_features, cat_features.astype(jnp.int32), cmax, emb0,
      l1_hW, row2(l1_hb), l1_m1W, row2(l1_m1b), l1_m2W, row2(l1_m2b))
  f1 = _segsum_l1(src2d, dst2d, msg1.reshape(4 * N, 16), zeros_nf)
  hid2, msg2 = _layer1_post(
      f1, hid1, l1_a1W, row2(l1_a1b), l1_a2W, row2(l1_a2b),
      l2_hW, row2(l2_hb), l2_m1W, row2(l2_m1b), l2_m2W, row2(l2_m2b))
  f2p = _segsum_l2(src2d, dst2d, msg2, zeros_nf)
  return _layer2_post(f2p, hid2, l2_a1W, row2(l2_a1b), l2_a2W, row2(l2_a2b))


# speedup vs baseline: 1.2832x; 1.2832x over previous
import functools

import jax
import jax.numpy as jnp
from jax import lax
from jax.experimental import pallas as pl
from jax.experimental.pallas import tpu as pltpu
from jax.experimental.pallas import tpu_sc as plsc

N = 100000
E = 1600000
NR = N // 8
NRP = 12544
NP = NRP * 8
K = 128
IDXB = 8
PAD_ROWS = 12544
EPAD = PAD_ROWS * K
NF = NP
NFR = NF // 8
BR = 448
GRID = NRP // BR


def _colmax_body(nsteps, x_ref, o_ref):
  i = pl.program_id(0)

  @pl.when(i == 0)
  def _():
    o_ref[...] = jnp.zeros_like(o_ref)

  m = jnp.max(jnp.abs(x_ref[:, 0:40]), axis=0, keepdims=True)
  m128 = jnp.concatenate(
      [jnp.broadcast_to(m, (8, 40)), jnp.zeros((8, 88), jnp.float32)], axis=1)
  acc = jnp.maximum(o_ref[...], m128)
  o_ref[...] = acc

  @pl.when(i == nsteps - 1)
  def _():
    m40 = jnp.max(acc, axis=0, keepdims=True)
    m5 = m40[:, 0:5]
    for j in range(1, 8):
      m5 = jnp.maximum(m5, m40[:, 5 * j:5 * j + 5])
    s5 = 1.0 / (m5 + 1e-12)
    s40 = jnp.concatenate([s5] * 8, axis=1)
    full = jnp.concatenate(
        [jnp.broadcast_to(s40, (8, 40)), jnp.zeros((8, 88), jnp.float32)],
        axis=1)
    o_ref[...] = full


def _colmax(x_p):
  return pl.pallas_call(
      functools.partial(_colmax_body, GRID),
      grid=(GRID,),
      in_specs=[pl.BlockSpec((BR, 128), lambda i: (i, 0))],
      out_specs=pl.BlockSpec((8, 128), lambda i: (0, 0)),
      out_shape=jax.ShapeDtypeStruct((8, 128), jnp.float32),
  )(x_p)


def _full(a):
  return pl.BlockSpec(a.shape, lambda i: tuple(0 for _ in a.shape))


def _l1pre_body(x_ref, sc_ref, r8_ref, hwr_ref, hwe_ref, hb_ref,
                m1_ref, m1b_ref, m2_ref, m2b_ref, hid_ref, msg_ref):
  xr = x_ref[:, 0:40] * sc_ref[0:1, 0:40]
  catrep = jnp.dot(x_ref[:, 40:48], r8_ref[...],
                   preferred_element_type=jnp.float32)
  lane16 = (lax.broadcasted_iota(jnp.int32, (BR, 128), 1) % 16
            ).astype(jnp.float32)
  onehot = (catrep == lane16).astype(jnp.float32)
  hid = jax.nn.relu(
      jnp.dot(xr, hwr_ref[...], preferred_element_type=jnp.float32)
      + jnp.dot(onehot, hwe_ref[...], preferred_element_type=jnp.float32)
      + hb_ref[...])
  t = jax.nn.relu(jnp.dot(hid, m1_ref[...],
                          preferred_element_type=jnp.float32) + m1b_ref[...])
  msg = jax.nn.relu(jnp.dot(t, m2_ref[...],
                            preferred_element_type=jnp.float32) + m2b_ref[...])
  hid_ref[...] = hid
  msg_ref[...] = msg


def _l1pre(x_p, scale, r8, hwr, hwe, hb, m1, m1b, m2, m2b):
  args = (scale, r8, hwr, hwe, hb, m1, m1b, m2, m2b)
  return pl.pallas_call(
      _l1pre_body,
      grid=(GRID,),
      in_specs=[
          pl.BlockSpec((BR, 128), lambda i: (i, 0)),
      ] + [_full(a) for a in args],
      out_specs=[
          pl.BlockSpec((BR, 512), lambda i: (i, 0)),
          pl.BlockSpec((BR, 512), lambda i: (i, 0)),
      ],
      out_shape=[
          jax.ShapeDtypeStruct((NRP, 512), jnp.float32),
          jax.ShapeDtypeStruct((NRP, 512), jnp.float32),
      ],
  )(x_p, *args)


def _l1post_body(f_ref, hid_ref, a10_ref, a11_ref, a12_ref, a13_ref, a1b_ref,
                 a2_ref, a2b_ref, hw_ref, hb_ref, m1_ref, m1b_ref,
                 m2_ref, m2b_ref, hid2_ref, msg2_ref):
  a1s = (a10_ref, a11_ref, a12_ref, a13_ref)
  acc = a1b_ref[...]
  for g in range(4):
    acc = acc + jnp.dot(f_ref[g], a1s[g][...],
                        preferred_element_type=jnp.float32)
  t = jax.nn.relu(acc)
  agg = jax.nn.relu(jnp.dot(t, a2_ref[...],
                            preferred_element_type=jnp.float32) + a2b_ref[...])
  x2 = agg + hid_ref[...]
  hid2 = jax.nn.relu(jnp.dot(x2, hw_ref[...],
                             preferred_element_type=jnp.float32) + hb_ref[...])
  t2 = jax.nn.relu(jnp.dot(hid2, m1_ref[...],
                           preferred_element_type=jnp.float32) + m1b_ref[...])
  msg2 = jax.nn.relu(jnp.dot(t2, m2_ref[...],
                             preferred_element_type=jnp.float32) + m2b_ref[...])
  hid2_ref[...] = hid2
  msg2_ref[...] = msg2


def _l1post(f1, hid1p, *ws):
  return pl.pallas_call(
      _l1post_body,
      grid=(GRID,),
      in_specs=[
          pl.BlockSpec((4, BR, 128), lambda i: (0, i, 0)),
          pl.BlockSpec((BR, 512), lambda i: (i, 0)),
      ] + [_full(a) for a in ws],
      out_specs=[
          pl.BlockSpec((BR, 128), lambda i: (i, 0)),
          pl.BlockSpec((BR, 128), lambda i: (i, 0)),
      ],
      out_shape=[
          jax.ShapeDtypeStruct((NRP, 128), jnp.float32),
          jax.ShapeDtypeStruct((NRP, 128), jnp.float32),
      ],
  )(f1, hid1p, *ws)


def _l2post_body(f_ref, hid_ref, a1_ref, a1b_ref, a2_ref, a2b_ref, out_ref):
  f = f_ref[0] + f_ref[1]
  t = jax.nn.relu(jnp.dot(f, a1_ref[...],
                          preferred_element_type=jnp.float32) + a1b_ref[...])
  agg = jax.nn.relu(jnp.dot(t, a2_ref[...],
                            preferred_element_type=jnp.float32) + a2b_ref[...])
  out_ref[...] = agg + hid_ref[...]


def _l2post(f2p, hid2p, *ws):
  return pl.pallas_call(
      _l2post_body,
      grid=(GRID,),
      in_specs=[
          pl.BlockSpec((2, BR, 128), lambda i: (0, i, 0)),
          pl.BlockSpec((BR, 128), lambda i: (i, 0)),
      ] + [_full(a) for a in ws],
      out_specs=pl.BlockSpec((BR, 128), lambda i: (i, 0)),
      out_shape=jax.ShapeDtypeStruct((NRP, 128), jnp.float32),
  )(f2p, hid2p, *ws)


def _make_segsum(n_out, n_msg, tasks_per_sc, rows_per_tile, msg_cols):
  n_sb = rows_per_tile // IDXB
  mesh = plsc.VectorSubcoreMesh(core_axis_name="c", subcore_axis_name="s")

  @functools.partial(
      pl.kernel,
      out_type=jax.ShapeDtypeStruct((n_out, NF, 16), jnp.float32),
      mesh=mesh,
      compiler_params=pltpu.CompilerParams(use_tc_tiling_on_sc=False),
      scratch_types=[
          pltpu.VMEM_SHARED((NF, 16), jnp.float32),
          pltpu.VMEM((IDXB, K), jnp.int32),
          pltpu.VMEM((IDXB, K), jnp.int32),
          pltpu.VMEM((IDXB, K), jnp.int32),
          pltpu.VMEM((2, K, 16), jnp.float32),
          pltpu.SemaphoreType.DMA,
      ],
  )
  def seg(src_hbm, dst_hbm, msg_hbm, zeros_hbm, out_hbm,
          fbuf, sidx, didx, gidx, rows, sem):
    c = lax.axis_index("c")
    s = lax.axis_index("s")
    zchunk = NF // 16
    msgv = msg_hbm
    for t in range(tasks_per_sc):
      if n_msg == 1:
        slot = c
        row0 = (c * 16 + s) * rows_per_tile
      else:
        slot = 2 * c + t
        row0 = s * rows_per_tile

      pltpu.sync_copy(zeros_hbm.at[pl.ds(s * zchunk, zchunk)],
                      fbuf.at[pl.ds(s * zchunk, zchunk)])
      plsc.subcore_barrier()

      @pl.loop(0, n_sb)
      def _(sb):
        rb = row0 + sb * IDXB
        pltpu.sync_copy(src_hbm.at[pl.ds(rb, IDXB)], sidx)
        pltpu.sync_copy(dst_hbm.at[pl.ds(rb, IDXB)], didx)
        if n_msg > 1:
          for j in range(IDXB):
            for q in range(K // 16):
              v = sidx[j, pl.ds(q * 16, 16)]
              gidx[j, pl.ds(q * 16, 16)] = v * n_msg + slot
          idxbuf = gidx
        else:
          idxbuf = sidx
        cps = [pltpu.async_copy(msgv.at[idxbuf.at[0]], rows.at[0], sem)]
        for j in range(IDXB):
          if j + 1 < IDXB:
            cps.append(pltpu.async_copy(msgv.at[idxbuf.at[j + 1]],
                                        rows.at[(j + 1) % 2], sem))
          cps[j].wait()
          pltpu.sync_copy(rows.at[j % 2], fbuf.at[didx.at[j]], add=True)

      plsc.subcore_barrier()
      pltpu.sync_copy(fbuf.at[pl.ds(s * zchunk, zchunk)],
                      out_hbm.at[slot].at[pl.ds(s * zchunk, zchunk)])
      plsc.subcore_barrier()

  return seg


_make_segsum = functools.cache(_make_segsum)


def _segsum_l1(*args):
  return _make_segsum(n_out=4, n_msg=4, tasks_per_sc=2,
                      rows_per_tile=PAD_ROWS // 16, msg_cols=512)(*args)


def _segsum_l2(*args):
  return _make_segsum(n_out=2, n_msg=1, tasks_per_sc=1,
                      rows_per_tile=PAD_ROWS // 32, msg_cols=128)(*args)


def kernel(real_features, cat_features, edge_index, emb0,
           l1_hW, l1_hb, l1_m1W, l1_m1b, l1_m2W, l1_m2b,
           l1_a1W, l1_a1b, l1_a2W, l1_a2b,
           l2_hW, l2_hb, l2_m1W, l2_m1b, l2_m2W, l2_m2b,
           l2_a1W, l2_a1b, l2_a2W, l2_a2b):
  f32 = jnp.float32
  i8 = jnp.eye(8, dtype=f32)
  bd = lambda w: jnp.kron(i8, w)
  t8 = lambda b: jnp.tile(b, 8).reshape(1, -1)

  rowpad = NRP - NR
  x_p = jnp.concatenate(
      [jnp.concatenate([real_features.reshape(NR, 40),
                        jnp.zeros((rowpad, 40), f32)]),
       jnp.concatenate([cat_features.reshape(NR, 8).astype(f32),
                        jnp.zeros((rowpad, 8), f32)]),
       jnp.zeros((NRP, 80), f32)], axis=1)
  r8 = jnp.kron(i8, jnp.ones((1, 16), f32))

  src = edge_index[0].astype(jnp.int32)
  dst = edge_index[1].astype(jnp.int32)
  pad = EPAD - E
  src2d = jnp.concatenate([src, jnp.zeros((pad,), jnp.int32)]).reshape(
      PAD_ROWS, K)
  dst2d = jnp.concatenate([dst, jnp.full((pad,), N, jnp.int32)]).reshape(
      PAD_ROWS, K)
  zeros_nf = jnp.zeros((NF, 16), f32)

  scale = _colmax(x_p)
  hid1p, msg1p = _l1pre(
      x_p, scale, r8, bd(l1_hW[:5]), bd(emb0 @ l1_hW[5:]), t8(l1_hb),
      bd(l1_m1W), t8(l1_m1b), bd(l1_m2W), t8(l1_m2b))
  f1 = _segsum_l1(src2d, dst2d, msg1p.reshape(4 * NP, 16), zeros_nf)
  hid2p, msg2p = _l1post(
      f1.reshape(4, NFR, 128), hid1p,
      bd(l1_a1W[0:16]), bd(l1_a1W[16:32]), bd(l1_a1W[32:48]), bd(l1_a1W[48:64]),
      t8(l1_a1b), bd(l1_a2W), t8(l1_a2b),
      bd(l2_hW), t8(l2_hb), bd(l2_m1W), t8(l2_m1b), bd(l2_m2W), t8(l2_m2b))
  f2p = _segsum_l2(src2d, dst2d, msg2p.reshape(NP, 16), zeros_nf)
  outp = _l2post(f2p.reshape(2, NFR, 128), hid2p, bd(l2_a1W), t8(l2_a1b), bd(l2_a2W), t8(l2_a2b))
  return outp[:NR].reshape(N, 16)
